# baseline (speedup 1.0000x reference)
import functools

import jax
import jax.numpy as jnp
from jax import lax
from jax.experimental import pallas as pl
from jax.experimental.pallas import tpu as pltpu
from jax.experimental.pallas import tpu_sc as plsc

_NC = 2
_NS = 16
_NW = _NC * _NS
_LANES = 16
_CW = 128


def _pe_table(seq_len, dim):
    pos = jnp.arange(seq_len, dtype=jnp.float32)[:, None]
    i = jnp.arange(0, dim, 2, dtype=jnp.float32)
    div = jnp.exp(-(jnp.log(10000.0)) * i / dim)
    pe = jnp.zeros((seq_len, dim), dtype=jnp.float32)
    pe = pe.at[:, 0::2].set(jnp.sin(pos * div))
    pe = pe.at[:, 1::2].set(jnp.cos(pos * div))
    return pe


@functools.partial(jax.jit, static_argnums=(3, 4))
def _emb_lookup(idx, pe4, table, seq_len, nch):
    D = table.shape[1]
    mesh = plsc.VectorSubcoreMesh(core_axis_name="c", subcore_axis_name="s")

    @functools.partial(
        pl.kernel,
        mesh=mesh,
        compiler_params=pltpu.CompilerParams(use_tc_tiling_on_sc=False),
        out_type=jax.ShapeDtypeStruct((_NW, nch, _CW, D), jnp.float32),
        scratch_types=[
            pltpu.VMEM((nch, _CW), jnp.int32),
            pltpu.VMEM((4 * seq_len, D), jnp.float32),
            pltpu.VMEM((_CW, D), jnp.float32),
            pltpu.SemaphoreType.DMA,
        ],
    )
    def k(idx_hbm, pe4_hbm, table_hbm, out_hbm, idx_v, pe4_v, buf, sem):
        wid = lax.axis_index("s") * _NC + lax.axis_index("c")
        pltpu.sync_copy(idx_hbm.at[wid], idx_v)
        pltpu.sync_copy(pe4_hbm, pe4_v)

        def chunk(j, carry):
            pltpu.async_copy(table_hbm.at[idx_v.at[j]], buf, sem).wait()
            phase = lax.rem(j * _CW, seq_len)

            def row(r, c2):
                for c in range(D // _LANES):
                    sl = pl.ds(c * _LANES, _LANES)
                    buf[r, sl] = buf[r, sl] + pe4_v[phase + r, sl]
                return c2

            lax.fori_loop(0, _CW, row, 0, unroll=2)
            pltpu.sync_copy(buf, out_hbm.at[wid, j])
            return carry

        lax.fori_loop(0, nch, chunk, 0)

    return k(idx, pe4, table)


def kernel(sequences, token_table):
    B, L = sequences.shape
    V, D = token_table.shape
    R = B * L
    assert R % (_NW * _CW) == 0
    nch = R // (_NW * _CW)

    idx = sequences.reshape(_NW, nch, _CW).astype(jnp.int32)
    pe = _pe_table(L, D)
    pe4 = jnp.concatenate([pe, pe, pe, pe], axis=0)
    out = _emb_lookup(idx, pe4, token_table, L, nch)
    return out.reshape(B, L, D)


# speedup vs baseline: 1.0712x; 1.0712x over previous
import functools

import jax
import jax.numpy as jnp
from jax import lax
from jax.experimental import pallas as pl
from jax.experimental.pallas import tpu as pltpu
from jax.experimental.pallas import tpu_sc as plsc

_NC = 2
_NS = 16
_NW = _NC * _NS
_LANES = 16
_CW = 128


def _pe_table(seq_len, dim):
    pos = jnp.arange(seq_len, dtype=jnp.float32)[:, None]
    i = jnp.arange(0, dim, 2, dtype=jnp.float32)
    div = jnp.exp(-(jnp.log(10000.0)) * i / dim)
    pe = jnp.zeros((seq_len, dim), dtype=jnp.float32)
    pe = pe.at[:, 0::2].set(jnp.sin(pos * div))
    pe = pe.at[:, 1::2].set(jnp.cos(pos * div))
    return pe


@functools.partial(jax.jit, static_argnums=(3, 4))
def _emb_lookup(idx, pe4, table, seq_len, nch):
    D = table.shape[1]
    mesh = plsc.VectorSubcoreMesh(core_axis_name="c", subcore_axis_name="s")

    NBUF = 10
    H = 5

    @functools.partial(
        pl.kernel,
        mesh=mesh,
        compiler_params=pltpu.CompilerParams(use_tc_tiling_on_sc=False),
        out_type=jax.ShapeDtypeStruct((_NW, nch, _CW, D), jnp.float32),
        scratch_types=[
            pltpu.VMEM((nch, _CW), jnp.int32),
            pltpu.VMEM((4 * seq_len, D), jnp.float32),
            [pltpu.VMEM((_CW, D), jnp.float32) for _ in range(NBUF)],
            [pltpu.SemaphoreType.DMA for _ in range(NBUF)],
            [pltpu.SemaphoreType.DMA for _ in range(NBUF)],
        ],
    )
    def k(idx_hbm, pe4_hbm, table_hbm, out_hbm, idx_v, pe4_v, bufs, gsem, ssem):
        wid = lax.axis_index("s") * _NC + lax.axis_index("c")
        pltpu.sync_copy(idx_hbm.at[wid], idx_v)
        pltpu.sync_copy(pe4_hbm, pe4_v)

        def fire_gather(j):
            b = j % NBUF
            return pltpu.async_copy(table_hbm.at[idx_v.at[j]], bufs[b], gsem[b])

        def add_pe(j):
            buf = bufs[j % NBUF]
            phase = (j * _CW) % seq_len

            def row(r, c2):
                for c in range(D // _LANES):
                    sl = pl.ds(c * _LANES, _LANES)
                    buf[r, sl] = buf[r, sl] + pe4_v[phase + r, sl]
                return c2

            lax.fori_loop(0, _CW, row, 0, unroll=2)

        gh = [None] * nch
        sh = [None] * nch
        for j in range(H):
            gh[j] = fire_gather(j)
        for j in range(nch):
            b = j % NBUF
            gh[j].wait()
            add_pe(j)
            sh[j] = pltpu.async_copy(bufs[b], out_hbm.at[wid, j], ssem[b])
            jn = j + H
            if jn < nch:
                if jn - NBUF >= 0:
                    sh[jn - NBUF].wait()
                gh[jn] = fire_gather(jn)
        for j in range(max(0, nch - NBUF), nch):
            sh[j].wait()

    return k(idx, pe4, table)


def kernel(sequences, token_table):
    B, L = sequences.shape
    V, D = token_table.shape
    R = B * L
    assert R % (_NW * _CW) == 0
    nch = R // (_NW * _CW)

    idx = sequences.reshape(_NW, nch, _CW).astype(jnp.int32)
    pe = _pe_table(L, D)
    pe4 = jnp.concatenate([pe, pe, pe, pe], axis=0)
    out = _emb_lookup(idx, pe4, token_table, L, nch)
    return out.reshape(B, L, D)


# speedup vs baseline: 1.1098x; 1.0360x over previous
import functools

import jax
import jax.numpy as jnp
from jax import lax
from jax.experimental import pallas as pl
from jax.experimental.pallas import tpu as pltpu
from jax.experimental.pallas import tpu_sc as plsc

_NC = 2
_NS = 16
_NW = _NC * _NS
_LANES = 16
_CW = 128


def _pe_table(seq_len, dim):
    pos = jnp.arange(seq_len, dtype=jnp.float32)[:, None]
    i = jnp.arange(0, dim, 2, dtype=jnp.float32)
    div = jnp.exp(-(jnp.log(10000.0)) * i / dim)
    pe = jnp.zeros((seq_len, dim), dtype=jnp.float32)
    pe = pe.at[:, 0::2].set(jnp.sin(pos * div))
    pe = pe.at[:, 1::2].set(jnp.cos(pos * div))
    return pe


@functools.partial(jax.jit, static_argnums=(3, 4))
def _emb_lookup(idx, pe4, table, seq_len, nch):
    D = table.shape[1] // 2
    mesh = plsc.VectorSubcoreMesh(core_axis_name="c", subcore_axis_name="s")

    NBUF = 6
    H = 3

    @functools.partial(
        pl.kernel,
        mesh=mesh,
        compiler_params=pltpu.CompilerParams(use_tc_tiling_on_sc=False),
        out_type=jax.ShapeDtypeStruct((_NW * nch * _CW, D), jnp.float32),
        scratch_types=[
            pltpu.VMEM((nch, _CW), jnp.int32),
            pltpu.VMEM((4 * seq_len, D), jnp.float32),
            [pltpu.VMEM((_CW, 2 * D), jnp.float32) for _ in range(NBUF)],
            [pltpu.SemaphoreType.DMA for _ in range(NBUF)],
            [pltpu.SemaphoreType.DMA for _ in range(NBUF)],
        ],
    )
    def k(idx_hbm, pe4_hbm, table_hbm, out_hbm, idx_v, pe4_v, bufs, gsem, ssem):
        wid = lax.axis_index("s") * _NC + lax.axis_index("c")
        pltpu.sync_copy(idx_hbm.at[wid], idx_v)
        pltpu.sync_copy(pe4_hbm, pe4_v)

        def fire_gather(j):
            b = j % NBUF
            return pltpu.async_copy(table_hbm.at[idx_v.at[j]], bufs[b], gsem[b])

        def add_pe(j):
            buf = bufs[j % NBUF]
            phase = (j * _CW) % seq_len

            def row(r, c2):
                for c in range(D // _LANES):
                    sl = pl.ds(c * _LANES, _LANES)
                    buf[r, sl] = buf[r, sl] + pe4_v[phase + r, sl]
                return c2

            lax.fori_loop(0, _CW, row, 0, unroll=2)

        gh = [None] * nch
        sh = [None] * nch
        for j in range(H):
            gh[j] = fire_gather(j)
        for j in range(nch):
            b = j % NBUF
            gh[j].wait()
            add_pe(j)
            sh[j] = pltpu.async_copy(
                bufs[b].at[:, pl.ds(0, D)],
                out_hbm.at[pl.ds((wid * nch + j) * _CW, _CW)],
                ssem[b],
            )
            jn = j + H
            if jn < nch:
                if jn - NBUF >= 0:
                    sh[jn - NBUF].wait()
                gh[jn] = fire_gather(jn)
        for j in range(max(0, nch - NBUF), nch):
            sh[j].wait()

    return k(idx, pe4, table)


def kernel(sequences, token_table):
    B, L = sequences.shape
    V, D = token_table.shape
    R = B * L
    assert R % (_NW * _CW) == 0
    nch = R // (_NW * _CW)

    idx = sequences.reshape(_NW, nch, _CW).astype(jnp.int32)
    pe = _pe_table(L, D)
    pe4 = jnp.concatenate([pe, pe, pe, pe], axis=0)
    table_padded = jnp.pad(token_table, ((0, 0), (0, D)))
    out = _emb_lookup(idx, pe4, table_padded, L, nch)
    return out.reshape(B, L, D)
